# Initial kernel scaffold; baseline (speedup 1.0000x reference)
import jax
import jax.numpy as jnp
from jax.experimental import pallas as pl


def kernel(hidden, coords, mask, W_e1, b_e1, W_e2, b_e2, W_c1, b_c1, W_c2, b_c2, W_n1, b_n1, W_n2, b_n2, ln_g, ln_b):
    raise NotImplementedError("write your pallas kernel here")


# speedup vs baseline: 10.3299x; 10.3299x over previous
import functools

import jax
import jax.numpy as jnp
from jax import lax
from jax.experimental import pallas as pl
from jax.experimental.pallas import tpu as pltpu
from jax.experimental.pallas import tpu_sc as plsc

H = 256
KNN = 16
RAD = 2
STEP = 0.1
LN_EPS = 1e-5
PADR = 8
F32 = jnp.float32


def _silu(x):
    return x * jax.nn.sigmoid(x)


def _topk_body(n, rt, x_ref, xt_ref, nn_ref):
    r = pl.program_id(1)
    xb = x_ref[...]
    xt = xt_ref[...]
    rows = r * rt + lax.broadcasted_iota(jnp.int32, (rt, n), 0)
    cols = lax.broadcasted_iota(jnp.int32, (rt, n), 1)
    d2 = jnp.zeros((rt, n), F32)
    for c in range(3):
        diff = xb[:, c:c + 1] - xt[c:c + 1, :]
        d2 = d2 + diff * diff
    dist = jnp.sqrt(jnp.maximum(d2, 0.0))
    dist = jnp.where(rows == cols, jnp.inf, dist)
    outs = []
    for _ in range(KNN):
        m = jnp.min(dist, axis=1, keepdims=True)
        am = jnp.min(jnp.where(dist == m, cols, n), axis=1, keepdims=True)
        outs.append(am)
        dist = jnp.where(cols == am, jnp.inf, dist)
    nn_ref[...] = jnp.concatenate(outs, axis=1)


def _topk_call(coords, rt=128, interpret=False):
    b, n, _ = coords.shape
    xt = jnp.transpose(coords, (0, 2, 1))
    grid = (b, n // rt)
    return pl.pallas_call(
        functools.partial(_topk_body, n, rt),
        grid=grid,
        in_specs=[
            pl.BlockSpec((None, rt, 3), lambda bb, rr: (bb, rr, 0)),
            pl.BlockSpec((None, 3, n), lambda bb, rr: (bb, 0, 0)),
        ],
        out_specs=pl.BlockSpec((None, rt, KNN), lambda bb, rr: (bb, rr, 0)),
        out_shape=jax.ShapeDtypeStruct((b, n, KNN), jnp.int32),
        interpret=interpret,
    )(coords, xt)


def _pre_body(h_ref, ws_ref, wdst_ref, asrc_ref, adst_ref):
    hh = h_ref[...]
    asrc_ref[...] = jnp.dot(hh, ws_ref[...], preferred_element_type=F32)
    adst_ref[...] = jnp.dot(hh, wdst_ref[...], preferred_element_type=F32)


def _pre_call(hidden, w_src, w_dst, interpret=False):
    b, n, _ = hidden.shape
    return pl.pallas_call(
        _pre_body,
        grid=(b,),
        in_specs=[
            pl.BlockSpec((None, n, H), lambda bb: (bb, 0, 0)),
            pl.BlockSpec((H, H), lambda bb: (0, 0)),
            pl.BlockSpec((H, H), lambda bb: (0, 0)),
        ],
        out_specs=[
            pl.BlockSpec((None, n, H), lambda bb: (bb, 0, 0)),
            pl.BlockSpec((None, n, H), lambda bb: (bb, 0, 0)),
        ],
        out_shape=[
            jax.ShapeDtypeStruct((b, n, H), F32),
            jax.ShapeDtypeStruct((b, n, H), F32),
        ],
        interpret=interpret,
    )(hidden, w_src, w_dst)


GW = H + 128


def _gather_call(idx, tab):
    nw, nch, ch = idx.shape
    e_total = nw * nch * ch
    mesh = plsc.VectorSubcoreMesh(core_axis_name="c", subcore_axis_name="s")

    @functools.partial(
        pl.kernel,
        out_type=jax.ShapeDtypeStruct((e_total, GW), F32),
        mesh=mesh,
        scratch_types=[
            pltpu.VMEM((nch, ch), jnp.int32),
            pltpu.VMEM((ch, GW), F32),
            pltpu.VMEM((ch, GW), F32),
            pltpu.SemaphoreType.DMA,
            pltpu.SemaphoreType.DMA,
        ],
    )
    def gk(idx_hbm, tab_hbm, g_out, idx_v, buf0, buf1, s0, s1):
        wid = lax.axis_index("s") * mesh.num_cores + lax.axis_index("c")
        base = wid * (nch * ch)
        pltpu.sync_copy(idx_hbm.at[wid], idx_v)
        bufs = (buf0, buf1)
        sems = (s0, s1)

        def start(c):
            s = c % 2
            return pltpu.async_copy(tab_hbm.at[idx_v.at[c]], bufs[s], sems[s])

        pend = start(0)
        for c in range(nch):
            cur = pend
            if c + 1 < nch:
                pend = start(c + 1)
            cur.wait()
            pltpu.sync_copy(bufs[c % 2], g_out.at[pl.ds(base + c * ch, ch)])

    return gk(idx, tab)


def _edge_body(n, r, h_ref, x_ref, asrc_ref, adstp_ref, xp_ref, g_ref,
               nn_ref, be1_ref, wd_ref, we2_ref, be2_ref, wc1_ref, bc1_ref,
               wc2r_ref, bc2_ref, wn1_ref, bn1_ref, wn2_ref, bn2_ref,
               lng_ref, lnb_ref, ho_ref, xo_ref):
    rr = pl.program_id(1)
    r0 = rr * r
    hh = h_ref[...]
    x = x_ref[...]
    asrc = asrc_ref[...]
    be1 = be1_ref[...]
    wd = wd_ref[...]

    gx = g_ref[...]
    g = gx[:, :H].reshape(r, KNN, H)
    xg3 = gx[:, H:H + 3].reshape(r, KNN, 3)
    rel_k = x[:, None, :] - xg3
    d2_k = jnp.sum(rel_k * rel_k, axis=-1)
    nn = nn_ref[...]
    irow = r0 + lax.broadcasted_iota(jnp.int32, (r, KNN), 0)
    w_k = (jnp.abs(nn - irow) > RAD).astype(F32)
    t1_k = (g + asrc[:, None, :] + d2_k[:, :, None] * wd[None, :, :]
            + be1[None, :, :])

    offs = (-2, -1, 1, 2)
    iv = r0 + lax.broadcasted_iota(jnp.int32, (r, 1), 0)
    win_a = adstp_ref[pl.ds(r0, r + 2 * PADR), :]
    win_x = xp_ref[pl.ds(r0, r + 2 * PADR), :]
    seq_t1, seq_rel, seq_w = [], [], []
    for off in offs:
        adst_o = win_a[PADR + off:PADR + off + r]
        x_o = win_x[PADR + off:PADR + off + r]
        relo = x - x_o
        d2o = jnp.sum(relo * relo, axis=-1, keepdims=True)
        seq_t1.append(asrc + adst_o + d2o * wd + be1)
        seq_rel.append(relo)
        valid = (iv + off >= 0) & (iv + off < n)
        seq_w.append(valid.astype(F32))

    t1 = jnp.concatenate([t1_k.reshape(r * KNN, H)] + seq_t1, axis=0)
    a1 = _silu(t1)
    msg = _silu(jnp.dot(a1, we2_ref[...], preferred_element_type=F32)
                + be2_ref[...])
    c1 = _silu(jnp.dot(msg, wc1_ref[...], preferred_element_type=F32)
               + bc1_ref[...])
    coef = jnp.tanh(jnp.sum(c1 * wc2r_ref[...], axis=-1, keepdims=True)
                    + bc2_ref[...])

    msg_k = msg[:r * KNN].reshape(r, KNN, H)
    coef_k = coef[:r * KNN].reshape(r, KNN, 1)
    wk3 = w_k[:, :, None]
    agg_msg = jnp.sum(msg_k * wk3, axis=1)
    agg_delta = jnp.sum(rel_k * (coef_k * wk3), axis=1)
    deg = jnp.sum(w_k, axis=1, keepdims=True)
    for j, off in enumerate(offs):
        m_o = msg[r * KNN + j * r: r * KNN + (j + 1) * r]
        c_o = coef[r * KNN + j * r: r * KNN + (j + 1) * r]
        w_o = seq_w[j]
        agg_msg = agg_msg + m_o * w_o
        agg_delta = agg_delta + seq_rel[j] * (c_o * w_o)
        deg = deg + w_o

    x_new = x + STEP * agg_delta / jnp.maximum(deg, 1.0)

    nin = jnp.concatenate([hh, agg_msg], axis=-1)
    h1 = _silu(jnp.dot(nin, wn1_ref[...], preferred_element_type=F32)
               + bn1_ref[...])
    h2 = jnp.dot(h1, wn2_ref[...], preferred_element_type=F32) + bn2_ref[...]
    hn = hh + h2
    mu = jnp.mean(hn, axis=-1, keepdims=True)
    var = jnp.mean((hn - mu) ** 2, axis=-1, keepdims=True)
    hn = (hn - mu) / jnp.sqrt(var + LN_EPS) * lng_ref[...] + lnb_ref[...]
    ho_ref[...] = hn
    xo_ref[...] = x_new


def _edge_call(hidden, coords, a_src, adst_pad, x_pad, g, nn,
               be1, wd, w_e2, be2, w_c1, bc1, wc2r, bc2,
               w_n1, bn1, w_n2, bn2, lng, lnb, r=128, interpret=False):
    b, n, _ = hidden.shape
    npad = n + 2 * PADR
    grid = (b, n // r)

    def blk(shape, imap):
        return pl.BlockSpec(shape, imap)

    full2 = lambda s0, s1: pl.BlockSpec((s0, s1), lambda bb, rr: (0, 0))
    in_specs = [
        blk((None, r, H), lambda bb, rr: (bb, rr, 0)),
        blk((None, r, 3), lambda bb, rr: (bb, rr, 0)),
        blk((None, r, H), lambda bb, rr: (bb, rr, 0)),
        blk((None, npad, H), lambda bb, rr: (bb, 0, 0)),
        blk((None, npad, 3), lambda bb, rr: (bb, 0, 0)),
        blk((None, r * KNN, GW), lambda bb, rr: (bb, rr, 0)),
        blk((None, r, KNN), lambda bb, rr: (bb, rr, 0)),
        full2(1, H),
        full2(1, H),
        full2(H, H),
        full2(1, H),
        full2(H, H),
        full2(1, H),
        full2(1, H),
        full2(1, 1),
        full2(2 * H, H),
        full2(1, H),
        full2(H, H),
        full2(1, H),
        full2(1, H),
        full2(1, H),
    ]
    out_specs = [
        blk((None, r, H), lambda bb, rr: (bb, rr, 0)),
        blk((None, r, 3), lambda bb, rr: (bb, rr, 0)),
    ]
    return pl.pallas_call(
        functools.partial(_edge_body, n, r),
        grid=grid,
        in_specs=in_specs,
        out_specs=out_specs,
        out_shape=[
            jax.ShapeDtypeStruct((b, n, H), F32),
            jax.ShapeDtypeStruct((b, n, 3), F32),
        ],
        interpret=interpret,
    )(hidden, coords, a_src, adst_pad, x_pad, g, nn,
      be1, wd, w_e2, be2, w_c1, bc1, wc2r, bc2,
      w_n1, bn1, w_n2, bn2, lng, lnb)


def kernel(hidden, coords, mask, W_e1, b_e1, W_e2, b_e2, W_c1, b_c1,
           W_c2, b_c2, W_n1, b_n1, W_n2, b_n2, ln_g, ln_b):
    b, n, _ = hidden.shape
    nw, ch = 32, 128
    nch = (b * n * KNN) // (nw * ch)

    w_src = W_e1[:H]
    w_dst = W_e1[H:2 * H]
    wd = W_e1[2 * H].reshape(1, H)

    nn = _topk_call(coords)
    a_src, a_dst = _pre_call(hidden, w_src, w_dst)

    boff = (jnp.arange(b, dtype=jnp.int32) * n)[:, None, None]
    idx = (nn + boff).reshape(nw, nch, ch)
    x128 = jnp.pad(coords.reshape(b * n, 3), ((0, 0), (0, 125)))
    tab = jnp.concatenate([a_dst.reshape(b * n, H), x128], axis=1)
    g = _gather_call(idx, tab)

    adst_pad = jnp.pad(a_dst, ((0, 0), (PADR, PADR), (0, 0)))
    x_pad = jnp.pad(coords, ((0, 0), (PADR, PADR), (0, 0)))

    out_h, out_x = _edge_call(
        hidden, coords, a_src, adst_pad, x_pad,
        g.reshape(b, n * KNN, GW), nn,
        b_e1.reshape(1, H), wd, W_e2, b_e2.reshape(1, H),
        W_c1, b_c1.reshape(1, H), W_c2.reshape(1, H), b_c2.reshape(1, 1),
        W_n1, b_n1.reshape(1, H), W_n2, b_n2.reshape(1, H),
        ln_g.reshape(1, H), ln_b.reshape(1, H))
    return out_h, out_x


# speedup vs baseline: 15.6344x; 1.5135x over previous
import functools

import jax
import jax.numpy as jnp
from jax import lax
from jax.experimental import pallas as pl
from jax.experimental.pallas import tpu as pltpu
from jax.experimental.pallas import tpu_sc as plsc

H = 256
KNN = 16
RAD = 2
STEP = 0.1
LN_EPS = 1e-5
PADR = 8
F32 = jnp.float32
BF16 = jnp.bfloat16
GW = H // 2 + 128
HB = H // 2


def _silu(x):
    h = 0.5 * x
    return h + h * jnp.tanh(h)


def _build_body(n, rt, x_ref, xt_ref, h_ref, ws_ref, wdst_ref, be1_ref,
                nn_ref, asrc_ref, adstp_ref, xp_ref, tab_ref):
    rr = pl.program_id(0)
    r0 = rr * rt
    xb = x_ref[...]
    xt = xt_ref[...]
    rows = r0 + lax.broadcasted_iota(jnp.int32, (rt, n), 0)
    cols = lax.broadcasted_iota(jnp.int32, (rt, n), 1)
    d2 = jnp.zeros((rt, n), F32)
    for c in range(3):
        diff = xb[:, c:c + 1] - xt[c:c + 1, :]
        d2 = d2 + diff * diff
    dist = jnp.sqrt(jnp.maximum(d2, 0.0))
    colsf = cols.astype(F32)
    big = jnp.float32(2.0e9)
    dist = jnp.where(rows == cols, jnp.inf, dist)
    outs = []
    for _ in range(KNN):
        m = jnp.min(dist, axis=1, keepdims=True)
        am = jnp.min(jnp.where(dist == m, colsf, big), axis=1, keepdims=True)
        outs.append(am)
        dist = jnp.where(colsf == am, jnp.inf, dist)
    nn_ref[...] = jnp.concatenate(outs, axis=1).astype(jnp.int32)

    hh = h_ref[...]
    asrc_ref[...] = (jnp.dot(hh, ws_ref[...], preferred_element_type=F32)
                     + be1_ref[...])
    adst = jnp.dot(hh, wdst_ref[...], preferred_element_type=F32)

    @pl.when(rr == 0)
    def _():
        adstp_ref[0:PADR] = jnp.zeros((PADR, H), F32)
        adstp_ref[PADR + n:PADR + n + PADR] = jnp.zeros((PADR, H), F32)
        xp_ref[0:PADR] = jnp.zeros((PADR, 3), F32)
        xp_ref[PADR + n:PADR + n + PADR] = jnp.zeros((PADR, 3), F32)

    adstp_ref[pl.ds(PADR + r0, rt), :] = adst
    xp_ref[pl.ds(PADR + r0, rt), :] = xb
    au = lax.bitcast_convert_type(adst, jnp.uint32)
    rne = au + 0x7FFF + ((au >> 16) & 1)
    hi = rne[:, HB:] & jnp.uint32(0xFFFF0000)
    lo = rne[:, :HB] >> 16
    tab_ref[:, 0:HB] = lax.bitcast_convert_type(hi | lo, F32)
    tab_ref[:, HB:HB + 3] = xb


def _build_call(x, xt, h, w_src, w_dst, be1, rt=512, interpret=False):
    n = x.shape[0]
    npad = n + 2 * PADR
    grid = (n // rt,)
    return pl.pallas_call(
        functools.partial(_build_body, n, rt),
        grid=grid,
        in_specs=[
            pl.BlockSpec((rt, 3), lambda rr: (rr, 0)),
            pl.BlockSpec((3, n), lambda rr: (0, 0)),
            pl.BlockSpec((rt, H), lambda rr: (rr, 0)),
            pl.BlockSpec((H, H), lambda rr: (0, 0)),
            pl.BlockSpec((H, H), lambda rr: (0, 0)),
            pl.BlockSpec((1, H), lambda rr: (0, 0)),
        ],
        out_specs=[
            pl.BlockSpec((rt, KNN), lambda rr: (rr, 0)),
            pl.BlockSpec((rt, H), lambda rr: (rr, 0)),
            pl.BlockSpec((npad, H), lambda rr: (0, 0)),
            pl.BlockSpec((npad, 3), lambda rr: (0, 0)),
            pl.BlockSpec((rt, GW), lambda rr: (rr, 0)),
        ],
        out_shape=[
            jax.ShapeDtypeStruct((n, KNN), jnp.int32),
            jax.ShapeDtypeStruct((n, H), F32),
            jax.ShapeDtypeStruct((npad, H), F32),
            jax.ShapeDtypeStruct((npad, 3), F32),
            jax.ShapeDtypeStruct((n, GW), F32),
        ],
        interpret=interpret,
    )(x, xt, h, w_src, w_dst, be1)


def _gather_call(idx, tab):
    nw, nch, ch = idx.shape
    e_total = nw * nch * ch
    mesh = plsc.VectorSubcoreMesh(core_axis_name="c", subcore_axis_name="s")

    @functools.partial(
        pl.kernel,
        out_type=jax.ShapeDtypeStruct((e_total, GW), F32),
        mesh=mesh,
        scratch_types=[
            pltpu.VMEM((nch, ch), jnp.int32),
            pltpu.VMEM((ch, GW), F32),
            pltpu.VMEM((ch, GW), F32),
            pltpu.SemaphoreType.DMA,
            pltpu.SemaphoreType.DMA,
        ],
    )
    def gk(idx_hbm, tab_hbm, g_out, idx_v, buf0, buf1, s0, s1):
        wid = lax.axis_index("s") * mesh.num_cores + lax.axis_index("c")
        base = wid * (nch * ch)
        pltpu.sync_copy(idx_hbm.at[wid], idx_v)
        bufs = (buf0, buf1)
        sems = (s0, s1)

        def start(c):
            s = c % 2
            return pltpu.async_copy(tab_hbm.at[idx_v.at[c]], bufs[s], sems[s])

        pend = start(0)
        for c in range(nch):
            cur = pend
            if c + 1 < nch:
                pend = start(c + 1)
            cur.wait()
            pltpu.sync_copy(bufs[c % 2], g_out.at[pl.ds(base + c * ch, ch)])

    return gk(idx, tab)


def _edge_body(n, r, h_ref, x_ref, asrc_ref, adstp_ref, xp_ref, g_ref,
               nn_ref, wd_ref, we2_ref, be2_ref, wc1_ref, bc1_ref,
               wc2r_ref, bc2_ref, wn1_ref, bn1_ref, wn2_ref, bn2_ref,
               lng_ref, lnb_ref, ho_ref, xo_ref):
    rr = pl.program_id(0)
    r0 = rr * r
    hh = h_ref[...]
    x = x_ref[...]
    asrc = asrc_ref[...]
    wd = wd_ref[...]

    gx = g_ref[...]
    pk = lax.bitcast_convert_type(gx[:, 0:HB], jnp.uint32)
    g_lo = lax.bitcast_convert_type(pk << 16, F32)
    g_hi = lax.bitcast_convert_type(pk & jnp.uint32(0xFFFF0000), F32)
    g = jnp.concatenate([g_lo, g_hi], axis=1).reshape(r, KNN, H)
    xg3 = gx[:, HB:HB + 3].reshape(r, KNN, 3)
    rel_k = x[:, None, :] - xg3
    d2_k = jnp.sum(rel_k * rel_k, axis=-1)
    nn = nn_ref[...]
    irow = r0 + lax.broadcasted_iota(jnp.int32, (r, KNN), 0)
    w_k = (jnp.abs(nn - irow) > RAD).astype(F32)
    t1_k = g + asrc[:, None, :] + d2_k[:, :, None] * wd[None, :, :]

    offs = (-2, -1, 1, 2)
    iv = r0 + lax.broadcasted_iota(jnp.int32, (r, 1), 0)
    win_a = adstp_ref[pl.ds(r0, r + 2 * PADR), :]
    win_x = xp_ref[pl.ds(r0, r + 2 * PADR), :]
    seq_t1, seq_rel, seq_w = [], [], []
    for off in offs:
        adst_o = win_a[PADR + off:PADR + off + r]
        x_o = win_x[PADR + off:PADR + off + r]
        relo = x - x_o
        d2o = jnp.sum(relo * relo, axis=-1, keepdims=True)
        seq_t1.append(asrc + adst_o + d2o * wd)
        seq_rel.append(relo)
        valid = (iv + off >= 0) & (iv + off < n)
        seq_w.append(valid.astype(F32))

    t1 = jnp.concatenate([t1_k.reshape(r * KNN, H)] + seq_t1, axis=0)
    a1 = _silu(t1).astype(BF16)
    msg = _silu(jnp.dot(a1, we2_ref[...], preferred_element_type=F32)
                + be2_ref[...])
    c1 = _silu(jnp.dot(msg.astype(BF16), wc1_ref[...],
                       preferred_element_type=F32)
               + bc1_ref[...])
    coef = jnp.tanh(jnp.sum(c1 * wc2r_ref[...], axis=-1, keepdims=True)
                    + bc2_ref[...])

    msg_k = msg[:r * KNN].reshape(r, KNN, H)
    coef_k = coef[:r * KNN].reshape(r, KNN, 1)
    wk3 = w_k[:, :, None]
    agg_msg = jnp.sum(msg_k * wk3, axis=1)
    agg_delta = jnp.sum(rel_k * (coef_k * wk3), axis=1)
    deg = jnp.sum(w_k, axis=1, keepdims=True)
    for j, off in enumerate(offs):
        m_o = msg[r * KNN + j * r: r * KNN + (j + 1) * r]
        c_o = coef[r * KNN + j * r: r * KNN + (j + 1) * r]
        w_o = seq_w[j]
        agg_msg = agg_msg + m_o * w_o
        agg_delta = agg_delta + seq_rel[j] * (c_o * w_o)
        deg = deg + w_o

    x_new = x + STEP * agg_delta / jnp.maximum(deg, 1.0)

    nin = jnp.concatenate([hh, agg_msg], axis=-1)
    h1 = _silu(jnp.dot(nin, wn1_ref[...], preferred_element_type=F32)
               + bn1_ref[...])
    h2 = jnp.dot(h1, wn2_ref[...], preferred_element_type=F32) + bn2_ref[...]
    hn = hh + h2
    mu = jnp.mean(hn, axis=-1, keepdims=True)
    var = jnp.mean((hn - mu) ** 2, axis=-1, keepdims=True)
    hn = (hn - mu) / jnp.sqrt(var + LN_EPS) * lng_ref[...] + lnb_ref[...]
    ho_ref[...] = hn
    xo_ref[...] = x_new


def _edge_call(h, x, a_src, adst_pad, x_pad, g, nn,
               wd, w_e2, be2, w_c1, bc1, wc2r, bc2,
               w_n1, bn1, w_n2, bn2, lng, lnb, r=512, interpret=False):
    n = h.shape[0]
    npad = n + 2 * PADR
    grid = (n // r,)

    def blk(shape, imap):
        return pl.BlockSpec(shape, imap)

    full2 = lambda s0, s1: pl.BlockSpec((s0, s1), lambda rr: (0, 0))
    in_specs = [
        blk((r, H), lambda rr: (rr, 0)),
        blk((r, 3), lambda rr: (rr, 0)),
        blk((r, H), lambda rr: (rr, 0)),
        full2(npad, H),
        full2(npad, 3),
        blk((r * KNN, GW), lambda rr: (rr, 0)),
        blk((r, KNN), lambda rr: (rr, 0)),
        full2(1, H),
        full2(H, H),
        full2(1, H),
        full2(H, H),
        full2(1, H),
        full2(1, H),
        full2(1, 1),
        full2(2 * H, H),
        full2(1, H),
        full2(H, H),
        full2(1, H),
        full2(1, H),
        full2(1, H),
    ]
    out_specs = [
        blk((r, H), lambda rr: (rr, 0)),
        blk((r, 3), lambda rr: (rr, 0)),
    ]
    return pl.pallas_call(
        functools.partial(_edge_body, n, r),
        grid=grid,
        in_specs=in_specs,
        out_specs=out_specs,
        out_shape=[
            jax.ShapeDtypeStruct((n, H), F32),
            jax.ShapeDtypeStruct((n, 3), F32),
        ],
        interpret=interpret,
    )(h, x, a_src, adst_pad, x_pad, g, nn,
      wd, w_e2, be2, w_c1, bc1, wc2r, bc2,
      w_n1, bn1, w_n2, bn2, lng, lnb)


def kernel(hidden, coords, mask, W_e1, b_e1, W_e2, b_e2, W_c1, b_c1,
           W_c2, b_c2, W_n1, b_n1, W_n2, b_n2, ln_g, ln_b):
    b, n, _ = hidden.shape
    nw, ch = 32, 128
    nch = (n * KNN) // (nw * ch)

    w_src = W_e1[:H]
    w_dst = W_e1[H:2 * H]
    wd = W_e1[2 * H].reshape(1, H)
    tail = (wd, W_e2.astype(BF16), b_e2.reshape(1, H),
            W_c1.astype(BF16), b_c1.reshape(1, H),
            W_c2.reshape(1, H), b_c2.reshape(1, 1),
            W_n1, b_n1.reshape(1, H), W_n2, b_n2.reshape(1, H),
            ln_g.reshape(1, H), ln_b.reshape(1, H))

    builds, gs = [], []
    for k in range(b):
        x_k = coords[k]
        bk = _build_call(x_k, x_k.T, hidden[k], w_src, w_dst,
                         b_e1.reshape(1, H))
        g_k = _gather_call(bk[0].reshape(nw, nch, ch), bk[4])
        builds.append(bk)
        gs.append(g_k)
    outs = []
    for k in range(b):
        nn_k, a_src, adst_pad, x_pad, _ = builds[k]
        outs.append(_edge_call(
            hidden[k], coords[k], a_src, adst_pad, x_pad,
            gs[k].reshape(n * KNN, GW), nn_k, *tail))
    out_h = jnp.stack([o[0] for o in outs], axis=0)
    out_x = jnp.stack([o[1] for o in outs], axis=0)
    return out_h, out_x
